# Initial kernel scaffold; baseline (speedup 1.0000x reference)
import jax
import jax.numpy as jnp
from jax.experimental import pallas as pl


def kernel(x, gate_W, W_gate, W_up, W_down, b_gate, b_up, b_down, Wr_gate, Wr_up, Wr_down, br_gate, br_up, br_down):
    raise NotImplementedError("write your pallas kernel here")


# speedup vs baseline: 1.5934x; 1.5934x over previous
import functools

import jax
import jax.numpy as jnp
from jax import lax
from jax.experimental import pallas as pl

K = 2


def _sigmoid(a):
    return 1.0 / (1.0 + jnp.exp(-a))


def _router_body(x_ref, gw_ref, comb_ref, *, E):
    x = x_ref[...]
    gw = gw_ref[...]
    logits = jnp.dot(x, gw, preferred_element_type=jnp.float32)
    m = jnp.max(logits, axis=-1, keepdims=True)
    ex = jnp.exp(logits - m)
    probs = ex / jnp.sum(ex, axis=-1, keepdims=True)
    iota = lax.broadcasted_iota(jnp.int32, probs.shape, 1)
    m1 = jnp.max(probs, axis=-1, keepdims=True)
    i1 = jnp.min(jnp.where(probs == m1, iota, E), axis=-1, keepdims=True)
    mask1 = iota == i1
    probs2 = jnp.where(mask1, -1.0, probs)
    m2 = jnp.max(probs2, axis=-1, keepdims=True)
    i2 = jnp.min(jnp.where(probs2 == m2, iota, E), axis=-1, keepdims=True)
    keep = mask1 | (iota == i2)
    comb_ref[...] = jnp.where(keep, probs, 0.0)


def _glu_body(x_ref, wg_ref, wu_ref, wd_ref, bg_ref, bu_ref, bd_ref,
              comb_ref, out_ref, *, BM):
    e = pl.program_id(0)
    mt = pl.program_id(1)
    x = x_ref[pl.ds(mt * BM, BM), :]
    wg = wg_ref[0]
    wu = wu_ref[0]
    wd = wd_ref[0]
    a = jnp.dot(x, wg, preferred_element_type=jnp.float32) + bg_ref[0]
    u = jnp.dot(x, wu, preferred_element_type=jnp.float32) + bu_ref[0]
    h = (a * _sigmoid(a)) * u
    y = jnp.dot(h, wd, preferred_element_type=jnp.float32) + bd_ref[0]
    c = comb_ref[0, 0][:, None]
    contrib = y * c

    @pl.when(e == 0)
    def _init():
        out_ref[pl.ds(mt * BM, BM), :] = contrib

    @pl.when(e != 0)
    def _acc():
        out_ref[pl.ds(mt * BM, BM), :] += contrib


def kernel(x, gate_W, W_gate, W_up, W_down, b_gate, b_up, b_down,
           Wr_gate, Wr_up, Wr_down, br_gate, br_up, br_down):
    T, D = x.shape
    E = gate_W.shape[1]
    HE = W_gate.shape[2]

    BT = min(T, 512)
    combine = pl.pallas_call(
        functools.partial(_router_body, E=E),
        grid=(T // BT,),
        in_specs=[
            pl.BlockSpec((BT, D), lambda i: (i, 0)),
            pl.BlockSpec((D, E), lambda i: (0, 0)),
        ],
        out_specs=pl.BlockSpec((BT, E), lambda i: (i, 0)),
        out_shape=jax.ShapeDtypeStruct((T, E), jnp.float32),
    )(x, gate_W)

    Wg_all = jnp.concatenate([W_gate, Wr_gate[None]], axis=0)
    Wu_all = jnp.concatenate([W_up, Wr_up[None]], axis=0)
    Wd_all = jnp.concatenate([W_down, Wr_down[None]], axis=0)
    bg_all = jnp.concatenate([b_gate, br_gate[None]], axis=0).reshape(E + 1, 1, HE)
    bu_all = jnp.concatenate([b_up, br_up[None]], axis=0).reshape(E + 1, 1, HE)
    bd_all = jnp.concatenate([b_down, br_down[None]], axis=0).reshape(E + 1, 1, D)
    comb3 = jnp.concatenate([combine, jnp.ones((T, 1), jnp.float32)],
                            axis=1).T.reshape(E + 1, 1, T)

    BM = min(T, 1024)
    MT = T // BM
    out = pl.pallas_call(
        functools.partial(_glu_body, BM=BM),
        grid=(E + 1, MT),
        in_specs=[
            pl.BlockSpec((T, D), lambda e, mt: (0, 0)),
            pl.BlockSpec((1, D, HE), lambda e, mt: (e, 0, 0)),
            pl.BlockSpec((1, D, HE), lambda e, mt: (e, 0, 0)),
            pl.BlockSpec((1, HE, D), lambda e, mt: (e, 0, 0)),
            pl.BlockSpec((1, 1, HE), lambda e, mt: (e, 0, 0)),
            pl.BlockSpec((1, 1, HE), lambda e, mt: (e, 0, 0)),
            pl.BlockSpec((1, 1, D), lambda e, mt: (e, 0, 0)),
            pl.BlockSpec((1, 1, BM), lambda e, mt: (e, 0, mt)),
        ],
        out_specs=pl.BlockSpec((T, D), lambda e, mt: (0, 0)),
        out_shape=jax.ShapeDtypeStruct((T, D), jnp.float32),
    )(x, Wg_all, Wu_all, Wd_all, bg_all, bu_all, bd_all, comb3)
    return out
